# baseline (speedup 1.0000x reference)
import functools

import jax
import jax.numpy as jnp
from jax import lax
from jax.experimental import pallas as pl
from jax.experimental.pallas import tpu as pltpu
from jax.experimental.pallas import tpu_sc as plsc

LANES = 16
NBUF = 4
BLK = 64
REDU = 4


def _bow_encode(idx_t, table, *, B, D, NC, NS):
    NW = NC * NS
    RPW = B // NW
    L = idx_t.shape[0]
    bounds = list(range(0, L, 128)) + [L]
    chunks = [(o, n - o) for o, n in zip(bounds[:-1], bounds[1:])]
    assert all(o % 8 == 0 and s % 8 == 0 and s <= 128 for o, s in chunks)
    DB = D // LANES
    NBLK = RPW // BLK
    TGRP = BLK // NBUF

    mesh = plsc.VectorSubcoreMesh(core_axis_name="c", subcore_axis_name="s")

    @functools.partial(
        pl.kernel,
        mesh=mesh,
        out_type=jax.ShapeDtypeStruct((B, D), jnp.float32),
        compiler_params=pltpu.CompilerParams(
            use_tc_tiling_on_sc=False, needs_layout_passes=False),
        scratch_types=[
            pltpu.VMEM((L, BLK), jnp.int32),
            pltpu.VMEM((BLK * L,), jnp.int32),
            pltpu.VMEM((NBUF, L, D), jnp.float32),
            pltpu.VMEM((BLK, D), jnp.float32),
        ]
        + [pltpu.SemaphoreType.DMA] * NBUF,
    )
    def run(idx_hbm, table_hbm, out_hbm, idx_tv, idx_v, rows_v, out_v, *sems):
        wid = lax.axis_index("s") * NC + lax.axis_index("c")
        base = wid * RPW

        def gather_row(r_local, b, make_only):
            mk = pltpu.make_async_copy if make_only else pltpu.async_copy
            return [
                mk(table_hbm.at[idx_v.at[pl.ds(r_local * L + o, s)]],
                   rows_v.at[b, pl.ds(o, s)], sems[b])
                for o, s in chunks
            ]

        def reduce_row(r_local, b):
            accs = tuple(rows_v[b, 0, pl.ds(LANES * d, LANES)]
                         for d in range(DB))

            def jbody(j, accs):
                out = []
                for d in range(DB):
                    a = accs[d]
                    for u in range(REDU):
                        a = jnp.maximum(
                            a, rows_v[b, j * REDU + u, pl.ds(LANES * d, LANES)])
                    out.append(a)
                return tuple(out)

            accs = lax.fori_loop(0, L // REDU, jbody, accs)
            for d in range(DB):
                out_v[r_local, pl.ds(LANES * d, LANES)] = accs[d]

        def transpose_idx():
            lanes = lax.iota(jnp.int32, LANES)

            def tbody(j, carry):
                for k in range(BLK // LANES):
                    v = idx_tv[j, pl.ds(LANES * k, LANES)]
                    plsc.store_scatter(
                        idx_v, [(lanes + LANES * k) * L + j], v + v)
                return carry

            lax.fori_loop(0, L, tbody, 0)

        def blk_body(blk, carry):
            blk_base = base + blk * BLK
            pltpu.sync_copy(idx_hbm.at[:, pl.ds(blk_base, BLK)], idx_tv)
            transpose_idx()
            for b in range(NBUF):
                gather_row(b, b, False)

            def grp_body(t, carry):
                for b in range(NBUF):
                    r = t * NBUF + b
                    for cp in gather_row(r, b, True):
                        cp.wait()
                    reduce_row(r, b)

                    @pl.when(t < TGRP - 1)
                    def _():
                        gather_row(r + NBUF, b, False)
                return carry

            lax.fori_loop(0, TGRP, grp_body, 0)
            pltpu.sync_copy(out_v, out_hbm.at[pl.ds(blk_base, BLK)])
            return carry

        lax.fori_loop(0, NBLK, blk_body, 0)

    return run(idx_t, table)


def kernel(input, emb_weight):
    B, L = input.shape
    V, D = emb_weight.shape
    NC, NS = 2, 16
    assert B % (NC * NS * BLK) == 0 and D % LANES == 0 and L % REDU == 0
    tab128 = jnp.concatenate(
        [emb_weight, jnp.zeros((V, 128 - D), emb_weight.dtype)], axis=1)
    tab2 = tab128.reshape(2 * V, D)
    return _bow_encode(input.T, tab2, B=B, D=D, NC=NC, NS=NS)


# speedup vs baseline: 1.0147x; 1.0147x over previous
import functools

import jax
import jax.numpy as jnp
from jax import lax
from jax.experimental import pallas as pl
from jax.experimental.pallas import tpu as pltpu
from jax.experimental.pallas import tpu_sc as plsc

LANES = 16
NBUF = 4
BLK = 128
REDU = 4


def _bow_encode(idx_t, table, *, B, D, NC, NS):
    NW = NC * NS
    RPW = B // NW
    L = idx_t.shape[0]
    bounds = list(range(0, L, 128)) + [L]
    chunks = [(o, n - o) for o, n in zip(bounds[:-1], bounds[1:])]
    assert all(o % 8 == 0 and s % 8 == 0 and s <= 128 for o, s in chunks)
    DB = D // LANES
    NBLK = RPW // BLK
    TGRP = BLK // NBUF

    mesh = plsc.VectorSubcoreMesh(core_axis_name="c", subcore_axis_name="s")

    @functools.partial(
        pl.kernel,
        mesh=mesh,
        out_type=jax.ShapeDtypeStruct((B, D), jnp.float32),
        compiler_params=pltpu.CompilerParams(
            use_tc_tiling_on_sc=False, needs_layout_passes=False),
        scratch_types=[
            pltpu.VMEM((L, BLK), jnp.int32),
            pltpu.VMEM((BLK * L,), jnp.int32),
            pltpu.VMEM((NBUF, L, D), jnp.float32),
            pltpu.VMEM((BLK, D), jnp.float32),
        ]
        + [pltpu.SemaphoreType.DMA] * NBUF,
    )
    def run(idx_hbm, table_hbm, out_hbm, idx_tv, idx_v, rows_v, out_v, *sems):
        wid = lax.axis_index("s") * NC + lax.axis_index("c")
        base = wid * RPW

        def gather_row(r_local, b, make_only):
            mk = pltpu.make_async_copy if make_only else pltpu.async_copy
            return [
                mk(table_hbm.at[idx_v.at[pl.ds(r_local * L + o, s)]],
                   rows_v.at[b, pl.ds(o, s)], sems[b])
                for o, s in chunks
            ]

        def reduce_row(r_local, b):
            accs = tuple(rows_v[b, 0, pl.ds(LANES * d, LANES)]
                         for d in range(DB))

            def jbody(j, accs):
                out = []
                for d in range(DB):
                    a = accs[d]
                    for u in range(REDU):
                        a = jnp.maximum(
                            a, rows_v[b, j * REDU + u, pl.ds(LANES * d, LANES)])
                    out.append(a)
                return tuple(out)

            accs = lax.fori_loop(0, L // REDU, jbody, accs)
            for d in range(DB):
                out_v[r_local, pl.ds(LANES * d, LANES)] = accs[d]

        def transpose_idx():
            lanes = lax.iota(jnp.int32, LANES)

            def tbody(j, carry):
                for k in range(BLK // LANES):
                    v = idx_tv[j, pl.ds(LANES * k, LANES)]
                    plsc.store_scatter(
                        idx_v, [(lanes + LANES * k) * L + j], v + v)
                return carry

            lax.fori_loop(0, L, tbody, 0)

        def blk_body(blk, carry):
            blk_base = base + blk * BLK
            pltpu.sync_copy(idx_hbm.at[:, pl.ds(blk_base, BLK)], idx_tv)
            transpose_idx()
            for b in range(NBUF):
                gather_row(b, b, False)

            def grp_body(t, carry):
                for b in range(NBUF):
                    r = t * NBUF + b
                    for cp in gather_row(r, b, True):
                        cp.wait()
                    reduce_row(r, b)

                    @pl.when(t < TGRP - 1)
                    def _():
                        gather_row(r + NBUF, b, False)
                return carry

            lax.fori_loop(0, TGRP, grp_body, 0)
            pltpu.sync_copy(out_v, out_hbm.at[pl.ds(blk_base, BLK)])
            return carry

        lax.fori_loop(0, NBLK, blk_body, 0)

    return run(idx_t, table)


def kernel(input, emb_weight):
    B, L = input.shape
    V, D = emb_weight.shape
    NC, NS = 2, 16
    assert B % (NC * NS * BLK) == 0 and D % LANES == 0 and L % REDU == 0
    tab128 = jnp.concatenate(
        [emb_weight, jnp.zeros((V, 128 - D), emb_weight.dtype)], axis=1)
    tab2 = tab128.reshape(2 * V, D)
    return _bow_encode(input.T, tab2, B=B, D=D, NC=NC, NS=NS)


# speedup vs baseline: 1.0175x; 1.0027x over previous
import functools

import jax
import jax.numpy as jnp
from jax import lax
from jax.experimental import pallas as pl
from jax.experimental.pallas import tpu as pltpu
from jax.experimental.pallas import tpu_sc as plsc

LANES = 16
NBUF = 4
BLK = 128
REDU = 8


def _bow_encode(idx_t, table, *, B, D, NC, NS):
    NW = NC * NS
    RPW = B // NW
    L = idx_t.shape[0]
    bounds = list(range(0, L, 128)) + [L]
    chunks = [(o, n - o) for o, n in zip(bounds[:-1], bounds[1:])]
    assert all(o % 8 == 0 and s % 8 == 0 and s <= 128 for o, s in chunks)
    DB = D // LANES
    NBLK = RPW // BLK
    TGRP = BLK // NBUF

    mesh = plsc.VectorSubcoreMesh(core_axis_name="c", subcore_axis_name="s")

    @functools.partial(
        pl.kernel,
        mesh=mesh,
        out_type=jax.ShapeDtypeStruct((B, D), jnp.float32),
        compiler_params=pltpu.CompilerParams(
            use_tc_tiling_on_sc=False, needs_layout_passes=False),
        scratch_types=[
            pltpu.VMEM((L, BLK), jnp.int32),
            pltpu.VMEM((BLK * L,), jnp.int32),
            pltpu.VMEM((NBUF, L, D), jnp.float32),
            pltpu.VMEM((BLK, D), jnp.float32),
        ]
        + [pltpu.SemaphoreType.DMA] * NBUF,
    )
    def run(idx_hbm, table_hbm, out_hbm, idx_tv, idx_v, rows_v, out_v, *sems):
        wid = lax.axis_index("s") * NC + lax.axis_index("c")
        base = wid * RPW

        def gather_row(r_local, b, make_only):
            mk = pltpu.make_async_copy if make_only else pltpu.async_copy
            return [
                mk(table_hbm.at[idx_v.at[pl.ds(r_local * L + o, s)]],
                   rows_v.at[b, pl.ds(o, s)], sems[b])
                for o, s in chunks
            ]

        def reduce_row(r_local, b):
            accs = tuple(rows_v[b, 0, pl.ds(LANES * d, LANES)]
                         for d in range(DB))

            def jbody(j, accs):
                out = []
                for d in range(DB):
                    a = accs[d]
                    for u in range(REDU):
                        a = jnp.maximum(
                            a, rows_v[b, j * REDU + u, pl.ds(LANES * d, LANES)])
                    out.append(a)
                return tuple(out)

            accs = lax.fori_loop(0, L // REDU, jbody, accs)
            for d in range(DB):
                out_v[r_local, pl.ds(LANES * d, LANES)] = accs[d]

        def transpose_idx():
            lanes = lax.iota(jnp.int32, LANES)

            def tbody(j, carry):
                for k in range(BLK // LANES):
                    v = idx_tv[j, pl.ds(LANES * k, LANES)]
                    plsc.store_scatter(
                        idx_v, [(lanes + LANES * k) * L + j], v + v)
                return carry

            lax.fori_loop(0, L, tbody, 0)

        def blk_body(blk, carry):
            blk_base = base + blk * BLK
            pltpu.sync_copy(idx_hbm.at[:, pl.ds(blk_base, BLK)], idx_tv)
            transpose_idx()
            for b in range(NBUF):
                gather_row(b, b, False)

            def grp_body(t, carry):
                for b in range(NBUF):
                    r = t * NBUF + b
                    for cp in gather_row(r, b, True):
                        cp.wait()
                    reduce_row(r, b)

                    @pl.when(t < TGRP - 1)
                    def _():
                        gather_row(r + NBUF, b, False)
                return carry

            lax.fori_loop(0, TGRP, grp_body, 0)
            pltpu.sync_copy(out_v, out_hbm.at[pl.ds(blk_base, BLK)])
            return carry

        lax.fori_loop(0, NBLK, blk_body, 0)

    return run(idx_t, table)


def kernel(input, emb_weight):
    B, L = input.shape
    V, D = emb_weight.shape
    NC, NS = 2, 16
    assert B % (NC * NS * BLK) == 0 and D % LANES == 0 and L % REDU == 0
    tab128 = jnp.concatenate(
        [emb_weight, jnp.zeros((V, 128 - D), emb_weight.dtype)], axis=1)
    tab2 = tab128.reshape(2 * V, D)
    return _bow_encode(input.T, tab2, B=B, D=D, NC=NC, NS=NS)
